# baseline (speedup 1.0000x reference)
import jax
import jax.numpy as jnp
from jax import lax
from jax.experimental import pallas as pl
from jax.experimental.pallas import tpu as pltpu
from jax.experimental.pallas import tpu_sc as plsc

_D = 32
_B = 16384
_NC = 2
_NS = 16
_NW = _NC * _NS
_BPW = _B // _NW
_CHUNK = 128
_NCHUNK = _BPW // _CHUNK
_UNROLL = 8


def _body(idx_hbm, msk_hbm, mu_hbm, lv_hbm, muu_hbm, lvu_hbm,
          mu_out, lv_out,
          idx_v, msk_v, muu_v, lvu_v, mu_rows, lv_rows, sem_mu, sem_lv):
    wid = lax.axis_index("s") * _NC + lax.axis_index("c")
    base = wid * _BPW

    pltpu.sync_copy(idx_hbm.at[wid], idx_v)
    mu_copies = [
        pltpu.async_copy(mu_hbm.at[idx_v.at[j]],
                         mu_rows.at[pl.ds(j * _CHUNK, _CHUNK)], sem_mu)
        for j in range(_NCHUNK)
    ]
    lv_copies = [
        pltpu.async_copy(lv_hbm.at[idx_v.at[j]],
                         lv_rows.at[pl.ds(j * _CHUNK, _CHUNK)], sem_lv)
        for j in range(_NCHUNK)
    ]
    pltpu.sync_copy(msk_hbm.at[wid], msk_v)
    pltpu.sync_copy(muu_hbm, muu_v)
    pltpu.sync_copy(lvu_hbm, lvu_v)
    mu_u = [muu_v[pl.ds(16 * t, 16)] for t in range(2)]
    lv_u = [lvu_v[pl.ds(16 * t, 16)] for t in range(2)]
    for c in mu_copies:
        c.wait()
    for c in lv_copies:
        c.wait()

    def blend(g, carry):
        m16 = msk_v[pl.ds(g * 16, 16)]
        for r in range(16):
            i = g * 16 + r
            keep = m16[r] != 0
            for t in range(2):
                sl = pl.ds(16 * t, 16)
                mu_rows[i, sl] = jnp.where(keep, mu_rows[i, sl], mu_u[t])
                lv_rows[i, sl] = jnp.where(keep, lv_rows[i, sl], lv_u[t])
        return carry

    lax.fori_loop(0, _BPW // 16, blend, 0)

    pltpu.sync_copy(mu_rows, mu_out.at[pl.ds(base, _BPW)])
    pltpu.sync_copy(lv_rows, lv_out.at[pl.ds(base, _BPW)])


def kernel(regime_id, regime_seen_mask, mu_embedding, logvar_embedding,
           mu_unknown, logvar_unknown):
    idx = regime_id.astype(jnp.int32).reshape(_NW, _NCHUNK, _CHUNK)
    msk = regime_seen_mask.astype(jnp.int32).reshape(_NW, _BPW)
    mesh = plsc.VectorSubcoreMesh(core_axis_name="c", subcore_axis_name="s")
    f = pl.kernel(
        _body,
        out_type=(jax.ShapeDtypeStruct((_B, _D), jnp.float32),
                  jax.ShapeDtypeStruct((_B, _D), jnp.float32)),
        mesh=mesh,
        compiler_params=pltpu.CompilerParams(use_tc_tiling_on_sc=False),
        scratch_types=[
            pltpu.VMEM((_NCHUNK, _CHUNK), jnp.int32),
            pltpu.VMEM((_BPW,), jnp.int32),
            pltpu.VMEM((_D,), jnp.float32),
            pltpu.VMEM((_D,), jnp.float32),
            pltpu.VMEM((_BPW, _D), jnp.float32),
            pltpu.VMEM((_BPW, _D), jnp.float32),
            pltpu.SemaphoreType.DMA,
            pltpu.SemaphoreType.DMA,
        ],
    )
    return f(idx, msk, mu_embedding, logvar_embedding,
             mu_unknown, logvar_unknown)


# speedup vs baseline: 1.0015x; 1.0015x over previous
import jax
import jax.numpy as jnp
from jax import lax
from jax.experimental import pallas as pl
from jax.experimental.pallas import tpu as pltpu
from jax.experimental.pallas import tpu_sc as plsc

_D = 32
_B = 16384
_NC = 2
_NS = 16
_NW = _NC * _NS
_BPW = _B // _NW
_CHUNK = 128
_NCHUNK = _BPW // _CHUNK


def _body(idx_hbm, msk_hbm, mu_hbm, lv_hbm, muu_hbm, lvu_hbm,
          mu_out, lv_out,
          idx_v, msk_v, muu_v, lvu_v, mu_rows, lv_rows, sem):
    wid = lax.axis_index("s") * _NC + lax.axis_index("c")
    base = wid * _BPW

    pltpu.sync_copy(idx_hbm.at[pl.ds(base, _BPW)], idx_v)
    copies = []
    for j in range(_NCHUNK):
        sl = pl.ds(j * _CHUNK, _CHUNK)
        copies.append(pltpu.async_copy(mu_hbm.at[idx_v.at[sl]],
                                       mu_rows.at[sl], sem))
        copies.append(pltpu.async_copy(lv_hbm.at[idx_v.at[sl]],
                                       lv_rows.at[sl], sem))
    pltpu.sync_copy(msk_hbm.at[pl.ds(base, _BPW)], msk_v)
    pltpu.sync_copy(muu_hbm, muu_v)
    pltpu.sync_copy(lvu_hbm, lvu_v)
    mu_u = [muu_v[pl.ds(16 * t, 16)] for t in range(2)]
    lv_u = [lvu_v[pl.ds(16 * t, 16)] for t in range(2)]
    for c in copies:
        c.wait()

    def blend(q, carry):
        m16 = msk_v[pl.ds(q * 16, 16)]
        for r in range(16):
            i = q * 16 + r
            keep = m16[r] != 0
            for t in range(2):
                sl = pl.ds(16 * t, 16)
                mu_rows[i, sl] = jnp.where(keep, mu_rows[i, sl], mu_u[t])
                lv_rows[i, sl] = jnp.where(keep, lv_rows[i, sl], lv_u[t])
        return carry

    lax.fori_loop(0, _BPW // 16, blend, 0)

    pltpu.sync_copy(mu_rows, mu_out.at[pl.ds(base, _BPW)])
    pltpu.sync_copy(lv_rows, lv_out.at[pl.ds(base, _BPW)])


def kernel(regime_id, regime_seen_mask, mu_embedding, logvar_embedding,
           mu_unknown, logvar_unknown):
    idx = regime_id.astype(jnp.int32)
    mesh = plsc.VectorSubcoreMesh(core_axis_name="c", subcore_axis_name="s")
    f = pl.kernel(
        _body,
        out_type=(jax.ShapeDtypeStruct((_B, _D), jnp.float32),
                  jax.ShapeDtypeStruct((_B, _D), jnp.float32)),
        mesh=mesh,
        compiler_params=pltpu.CompilerParams(use_tc_tiling_on_sc=False),
        scratch_types=[
            pltpu.VMEM((_BPW,), jnp.int32),
            pltpu.VMEM((_BPW,), jnp.int32),
            pltpu.VMEM((_D,), jnp.float32),
            pltpu.VMEM((_D,), jnp.float32),
            pltpu.VMEM((_BPW, _D), jnp.float32),
            pltpu.VMEM((_BPW, _D), jnp.float32),
            pltpu.SemaphoreType.DMA,
        ],
    )
    return f(idx, regime_seen_mask, mu_embedding, logvar_embedding,
             mu_unknown, logvar_unknown)
